# baseline (speedup 1.0000x reference)
import jax
import jax.numpy as jnp
from jax import lax
from jax.experimental import pallas as pl
from jax.experimental.pallas import tpu as pltpu
from jax.experimental.pallas import tpu_sc as plsc

N = 50000
E = 1600000
D_IN = 39
D_OUT = 64
DH = D_OUT // 2

NC = 2
NS = 16
L = 16

NPAD = 50176
PT = NPAD // NS

G = 128
GPT = 784
EPT = GPT * G
EPAD = EPT * NS
GPW = GPT // NC
NCH = GPT

ROWBLK = 512
NBLK = NPAD // ROWBLK
RB4 = 2000
NBLK4 = N // RB4


def _k1_body(dst2, degp, accum, idxbuf, ones, zbuf):
    c = lax.axis_index("c")
    s = lax.axis_index("s")

    def _zb(i, _):
        zbuf[pl.ds(i * L, L)] = jnp.zeros((L,), jnp.float32)
        return 0
    lax.fori_loop(0, PT // L, _zb, 0)

    def _ob(i, _):
        ones[pl.ds(i * L, L)] = jnp.ones((L,), jnp.float32)
        return 0
    lax.fori_loop(0, G // L, _ob, 0)

    pltpu.sync_copy(zbuf, accum.at[pl.ds(s * PT, PT)])
    plsc.subcore_barrier()

    pltpu.sync_copy(dst2.at[pl.ds(s * GPT + c * GPW, GPW)], idxbuf)

    def _grp(j, _):
        pltpu.sync_copy(ones, accum.at[idxbuf.at[j]], add=True)
        return 0
    lax.fori_loop(0, GPW, _grp, 0)

    plsc.subcore_barrier()
    pltpu.sync_copy(accum.at[pl.ds(s * PT, PT)], degp.at[c, pl.ds(s * PT, PT)])


_k1 = pl.kernel(
    _k1_body,
    out_type=jax.ShapeDtypeStruct((NC, NPAD), jnp.float32),
    mesh=plsc.VectorSubcoreMesh(core_axis_name="c", subcore_axis_name="s"),
    compiler_params=pltpu.CompilerParams(use_tc_tiling_on_sc=False),
    scratch_types=[
        pltpu.VMEM_SHARED((NPAD,), jnp.float32),
        pltpu.VMEM((GPW, G), jnp.int32),
        pltpu.VMEM((G,), jnp.float32),
        pltpu.VMEM((PT,), jnp.float32),
    ],
)


def _k2_body(x_ref, w_ref, deg_ref, hs_ref):
    deg = deg_ref[0] + deg_ref[1] + 1.0
    dis = lax.rsqrt(deg)
    h = jnp.dot(x_ref[...], w_ref[...], preferred_element_type=jnp.float32)
    hs = h * dis
    hs_ref[0] = hs[:, :DH]
    hs_ref[1] = hs[:, DH:]


def _k2(x, w, degp2):
    return pl.pallas_call(
        _k2_body,
        grid=(NBLK,),
        in_specs=[
            pl.BlockSpec((ROWBLK, D_IN), lambda i: (i, 0)),
            pl.BlockSpec((D_IN, D_OUT), lambda i: (0, 0)),
            pl.BlockSpec((NC, ROWBLK, 1), lambda i: (0, i, 0)),
        ],
        out_specs=pl.BlockSpec((NC, ROWBLK, DH), lambda i: (0, i, 0)),
        out_shape=jax.ShapeDtypeStruct((NC, NPAD, DH), jnp.float32),
    )(x, w, degp2)


def _k3_body(hs2, src2, dst2, agg, accum, srcbuf, dstbuf, rows, *sems):
    c = lax.axis_index("c")
    s = lax.axis_index("s")
    off = (c * NPAD).astype(jnp.int32)
    isems = sems[0:8]
    gsems = sems[8:12]
    ssems = sems[12:16]

    pltpu.sync_copy(hs2.at[pl.ds(c * NPAD + s * PT, PT)],
                    accum.at[pl.ds(s * PT, PT)])
    plsc.subcore_barrier()

    def _load_idx(g, w):
        row = s * GPT + g
        pltpu.async_copy(src2.at[pl.ds(row, 1)], srcbuf.at[pl.ds(w, 1)],
                         isems[w])
        pltpu.async_copy(dst2.at[pl.ds(row, 1)], dstbuf.at[pl.ds(w, 1)],
                         isems[w])

    def _drain_idx(w):
        pltpu.make_async_copy(src2.at[pl.ds(0, 1)], srcbuf.at[pl.ds(w, 1)],
                              isems[w]).wait()
        pltpu.make_async_copy(src2.at[pl.ds(0, 1)], dstbuf.at[pl.ds(w, 1)],
                              isems[w]).wait()

    def _bias(w):
        for k in range(G // L):
            sl = (w, pl.ds(k * L, L))
            srcbuf[sl] = srcbuf[sl] + off

    def _start_gather(w, q):
        pltpu.async_copy(hs2.at[srcbuf.at[w]], rows.at[q], gsems[q])

    def _drain_rows(sem, q):
        pltpu.make_async_copy(hs2.at[pl.ds(0, G)], rows.at[q], sem).wait()

    for g in range(6):
        _load_idx(g, g)
    for g in range(2):
        _drain_idx(g)
        _bias(g)
        _start_gather(g, g)

    def _step(i, _):
        g8 = i * 8
        for u in range(8):
            g = g8 + u
            q = u % 4
            _drain_rows(gsems[q], q)
            pltpu.async_copy(rows.at[q],
                             accum.at[dstbuf.at[u]], ssems[q], add=True)

            @pl.when(g >= 2)
            def _():
                _drain_rows(ssems[(u + 2) % 4], (u + 2) % 4)

            @pl.when(g + 6 < NCH)
            def _():
                _load_idx(g + 6, (u + 6) % 8)

            @pl.when(g + 2 < NCH)
            def _():
                _drain_idx((u + 2) % 8)
                _bias((u + 2) % 8)
                _start_gather((u + 2) % 8, (u + 2) % 4)
        return 0

    lax.fori_loop(0, NCH // 8, _step, 0)

    _drain_rows(ssems[(NCH - 2) % 4], (NCH - 2) % 4)
    _drain_rows(ssems[(NCH - 1) % 4], (NCH - 1) % 4)

    plsc.subcore_barrier()
    pltpu.sync_copy(accum.at[pl.ds(s * PT, PT)], agg.at[c, pl.ds(s * PT, PT)])


_k3 = pl.kernel(
    _k3_body,
    out_type=jax.ShapeDtypeStruct((NC, NPAD, DH), jnp.float32),
    mesh=plsc.VectorSubcoreMesh(core_axis_name="c", subcore_axis_name="s"),
    compiler_params=pltpu.CompilerParams(use_tc_tiling_on_sc=False),
    scratch_types=[
        pltpu.VMEM_SHARED((NPAD, DH), jnp.float32),
        pltpu.VMEM((8, G), jnp.int32),
        pltpu.VMEM((8, G), jnp.int32),
        pltpu.VMEM((4, G, DH), jnp.float32),
    ] + [pltpu.SemaphoreType.DMA] * 16,
)


def _k4_body(agg_ref, deg_ref, b_ref, out_ref):
    dis = lax.rsqrt(deg_ref[0] + deg_ref[1] + 1.0)
    out_ref[:, :DH] = agg_ref[0] * dis + b_ref[:, :DH]
    out_ref[:, DH:] = agg_ref[1] * dis + b_ref[:, DH:]


def _k4(agg, degp2, b2):
    return pl.pallas_call(
        _k4_body,
        grid=(NBLK4,),
        in_specs=[
            pl.BlockSpec((NC, RB4, DH), lambda i: (0, i, 0)),
            pl.BlockSpec((NC, RB4, 1), lambda i: (0, i, 0)),
            pl.BlockSpec((1, D_OUT), lambda i: (0, 0)),
        ],
        out_specs=pl.BlockSpec((RB4, D_OUT), lambda i: (i, 0)),
        out_shape=jax.ShapeDtypeStruct((N, D_OUT), jnp.float32),
    )(agg, degp2, b2)


@jax.jit
def kernel(x, edge_index, W, b):
    src = edge_index[0]
    dst = edge_index[1]
    npad_e = EPAD - E
    ar = jnp.arange(npad_e, dtype=jnp.int32)
    srcp = jnp.concatenate([src, ar % N])
    dstp = jnp.concatenate([dst, N + ar % (NPAD - N)])
    src2 = srcp.reshape(NS * GPT, G)
    dst2 = dstp.reshape(NS * GPT, G)

    b2 = b.reshape(1, D_OUT)

    degp = _k1(dst2)
    degp2 = degp.reshape(NC, NPAD, 1)
    hs = _k2(x, W, degp2)
    hs2 = hs.reshape(NC * NPAD, DH)
    agg = _k3(hs2, src2, dst2)
    out = _k4(agg, degp2, b2)
    return out


# speedup vs baseline: 1.0993x; 1.0993x over previous
import jax
import jax.numpy as jnp
from jax import lax
from jax.experimental import pallas as pl
from jax.experimental.pallas import tpu as pltpu
from jax.experimental.pallas import tpu_sc as plsc

N = 50000
E = 1600000
D_IN = 39
D_OUT = 64
DH = D_OUT // 2

NC = 2
NS = 16
L = 16

NPAD = 50176
PT = NPAD // NS

G = 128
GPT = 784
EPT = GPT * G
EPAD = EPT * NS
GPW = GPT // NC
CH = 2
NCH = GPT // CH
NSLOT = 4

ROWBLK = 512
NBLK = NPAD // ROWBLK
RB4 = 2000
NBLK4 = N // RB4


def _k1_body(dst2, degp, accum, idxbuf, ones, zbuf):
    c = lax.axis_index("c")
    s = lax.axis_index("s")

    def _zb(i, _):
        zbuf[pl.ds(i * L, L)] = jnp.zeros((L,), jnp.float32)
        return 0
    lax.fori_loop(0, PT // L, _zb, 0)

    def _ob(i, _):
        ones[pl.ds(i * L, L)] = jnp.ones((L,), jnp.float32)
        return 0
    lax.fori_loop(0, G // L, _ob, 0)

    pltpu.sync_copy(zbuf, accum.at[pl.ds(s * PT, PT)])
    plsc.subcore_barrier()

    pltpu.sync_copy(dst2.at[pl.ds(s * GPT + c * GPW, GPW)], idxbuf)

    def _grp(j, _):
        pltpu.sync_copy(ones, accum.at[idxbuf.at[j]], add=True)
        return 0
    lax.fori_loop(0, GPW, _grp, 0)

    plsc.subcore_barrier()
    pltpu.sync_copy(accum.at[pl.ds(s * PT, PT)], degp.at[c, pl.ds(s * PT, PT)])


_k1 = pl.kernel(
    _k1_body,
    out_type=jax.ShapeDtypeStruct((NC, NPAD), jnp.float32),
    mesh=plsc.VectorSubcoreMesh(core_axis_name="c", subcore_axis_name="s"),
    compiler_params=pltpu.CompilerParams(use_tc_tiling_on_sc=False),
    scratch_types=[
        pltpu.VMEM_SHARED((NPAD,), jnp.float32),
        pltpu.VMEM((GPW, G), jnp.int32),
        pltpu.VMEM((G,), jnp.float32),
        pltpu.VMEM((PT,), jnp.float32),
    ],
)


def _k2_body(x_ref, w_ref, deg_ref, hs_ref):
    deg = deg_ref[0] + deg_ref[1] + 1.0
    dis = lax.rsqrt(deg)
    h = jnp.dot(x_ref[...], w_ref[...], preferred_element_type=jnp.float32)
    hs = h * dis
    hs_ref[0] = hs[:, :DH]
    hs_ref[1] = hs[:, DH:]


def _k2(x, w, degp2):
    return pl.pallas_call(
        _k2_body,
        grid=(NBLK,),
        in_specs=[
            pl.BlockSpec((ROWBLK, D_IN), lambda i: (i, 0)),
            pl.BlockSpec((D_IN, D_OUT), lambda i: (0, 0)),
            pl.BlockSpec((NC, ROWBLK, 1), lambda i: (0, i, 0)),
        ],
        out_specs=pl.BlockSpec((NC, ROWBLK, DH), lambda i: (0, i, 0)),
        out_shape=jax.ShapeDtypeStruct((NC, NPAD, DH), jnp.float32),
    )(x, w, degp2)


def _k3_body(hs2, src2, dst2, agg, accum, srcbuf, dstbuf, rows,
             g0, g1, i0, i1, i2, i3, ssem):
    c = lax.axis_index("c")
    s = lax.axis_index("s")
    off = (c * NPAD).astype(jnp.int32)
    gsems = [g0, g1]
    isems = [i0, i1, i2, i3]

    pltpu.sync_copy(hs2.at[pl.ds(c * NPAD + s * PT, PT)],
                    accum.at[pl.ds(s * PT, PT)])
    plsc.subcore_barrier()

    def _load_idx(ch, q):
        pltpu.async_copy(src2.at[pl.ds(s * GPT + ch * CH, CH)], srcbuf.at[q],
                         isems[q])
        pltpu.async_copy(dst2.at[pl.ds(s * GPT + ch * CH, CH)], dstbuf.at[q],
                         isems[q])

    def _drain_idx(q):
        pltpu.make_async_copy(src2.at[pl.ds(0, CH)], srcbuf.at[q],
                              isems[q]).wait()
        pltpu.make_async_copy(src2.at[pl.ds(0, CH)], dstbuf.at[q],
                              isems[q]).wait()

    def _bias(q):
        for r in range(CH):
            for k in range(G // L):
                sl = (q, r, pl.ds(k * L, L))
                srcbuf[sl] = srcbuf[sl] + off

    def _start_gathers(b, q):
        for r in range(CH):
            pltpu.async_copy(hs2.at[srcbuf.at[q, r]], rows.at[b, r], gsems[b])

    def _drain_rows(sem, b):
        for r in range(CH):
            pltpu.make_async_copy(hs2.at[pl.ds(0, G)], rows.at[b, r],
                                  sem).wait()

    for q in range(NSLOT):
        _load_idx(q, q)
    for b in range(2):
        _drain_idx(b)
        _bias(b)
        _start_gathers(b, b)

    def _step(i, _):
        ch4 = i * NSLOT
        for u in range(NSLOT):
            ch = ch4 + u
            b = u % 2
            _drain_rows(gsems[b], b)
            for r in range(CH):
                pltpu.async_copy(rows.at[b, r],
                                 accum.at[dstbuf.at[u, r]], ssem, add=True)

            q2 = (u + 2) % NSLOT

            @pl.when(ch + 2 < NCH)
            def _():
                _drain_idx(q2)
                _bias(q2)

            _drain_rows(ssem, b)

            @pl.when(ch + NSLOT < NCH)
            def _():
                _load_idx(ch + NSLOT, u)

            @pl.when(ch + 2 < NCH)
            def _():
                _start_gathers(b, q2)
        return 0

    lax.fori_loop(0, NCH // NSLOT, _step, 0)

    plsc.subcore_barrier()
    pltpu.sync_copy(accum.at[pl.ds(s * PT, PT)], agg.at[c, pl.ds(s * PT, PT)])


_k3 = pl.kernel(
    _k3_body,
    out_type=jax.ShapeDtypeStruct((NC, NPAD, DH), jnp.float32),
    mesh=plsc.VectorSubcoreMesh(core_axis_name="c", subcore_axis_name="s"),
    compiler_params=pltpu.CompilerParams(use_tc_tiling_on_sc=False),
    scratch_types=[
        pltpu.VMEM_SHARED((NPAD, DH), jnp.float32),
        pltpu.VMEM((NSLOT, CH, G), jnp.int32),
        pltpu.VMEM((NSLOT, CH, G), jnp.int32),
        pltpu.VMEM((2, CH, G, DH), jnp.float32),
        pltpu.SemaphoreType.DMA,
        pltpu.SemaphoreType.DMA,
        pltpu.SemaphoreType.DMA,
        pltpu.SemaphoreType.DMA,
        pltpu.SemaphoreType.DMA,
        pltpu.SemaphoreType.DMA,
        pltpu.SemaphoreType.DMA,
    ],
)


def _k4_body(agg_ref, deg_ref, b_ref, out_ref):
    dis = lax.rsqrt(deg_ref[0] + deg_ref[1] + 1.0)
    out_ref[:, :DH] = agg_ref[0] * dis + b_ref[:, :DH]
    out_ref[:, DH:] = agg_ref[1] * dis + b_ref[:, DH:]


def _k4(agg, degp2, b2):
    return pl.pallas_call(
        _k4_body,
        grid=(NBLK4,),
        in_specs=[
            pl.BlockSpec((NC, RB4, DH), lambda i: (0, i, 0)),
            pl.BlockSpec((NC, RB4, 1), lambda i: (0, i, 0)),
            pl.BlockSpec((1, D_OUT), lambda i: (0, 0)),
        ],
        out_specs=pl.BlockSpec((RB4, D_OUT), lambda i: (i, 0)),
        out_shape=jax.ShapeDtypeStruct((N, D_OUT), jnp.float32),
    )(agg, degp2, b2)


@jax.jit
def kernel(x, edge_index, W, b):
    src = edge_index[0]
    dst = edge_index[1]
    npad_e = EPAD - E
    ar = jnp.arange(npad_e, dtype=jnp.int32)
    srcp = jnp.concatenate([src, ar % N])
    dstp = jnp.concatenate([dst, N + ar % (NPAD - N)])
    src2 = srcp.reshape(NS * GPT, G)
    dst2 = dstp.reshape(NS * GPT, G)

    b2 = b.reshape(1, D_OUT)

    degp = _k1(dst2)
    degp2 = degp.reshape(NC, NPAD, 1)
    hs = _k2(x, W, degp2)
    hs2 = hs.reshape(NC * NPAD, DH)
    agg = _k3(hs2, src2, dst2)
    out = _k4(agg, degp2, b2)
    return out


# speedup vs baseline: 1.1422x; 1.0391x over previous
import jax
import jax.numpy as jnp
from jax import lax
from jax.experimental import pallas as pl
from jax.experimental.pallas import tpu as pltpu
from jax.experimental.pallas import tpu_sc as plsc

N = 50000
E = 1600000
D_IN = 39
D_OUT = 64
DH = D_OUT // 2

NC = 2
NS = 16
L = 16

NPAD = 50176
PT = NPAD // NS

G = 128
GPT = 784
EPT = GPT * G
EPAD = EPT * NS
GPW = GPT // NC
CH = 2
NCH = GPT // CH
NSLOT = 4

ROWBLK = 512
NBLK = NPAD // ROWBLK
RB4 = 2000
NBLK4 = N // RB4


def _k1_body(dst2, degp, accum, idxbuf, ones, zbuf, h0, h1, h2, h3):
    c = lax.axis_index("c")
    s = lax.axis_index("s")
    hsems = [h0, h1, h2, h3]

    def _zb(i, _):
        zbuf[pl.ds(i * L, L)] = jnp.zeros((L,), jnp.float32)
        return 0
    lax.fori_loop(0, PT // L, _zb, 0)

    def _ob(i, _):
        ones[pl.ds(i * L, L)] = jnp.ones((L,), jnp.float32)
        return 0
    lax.fori_loop(0, G // L, _ob, 0)

    pltpu.sync_copy(zbuf, accum.at[pl.ds(s * PT, PT)])
    plsc.subcore_barrier()

    pltpu.sync_copy(dst2.at[pl.ds(s * GPT + c * GPW, GPW)], idxbuf)

    def _grp(i, _):
        for u in range(4):
            j = i * 4 + u

            @pl.when(i > 0)
            def _():
                pltpu.make_async_copy(ones, accum.at[pl.ds(0, G)],
                                      hsems[u]).wait()

            pltpu.async_copy(ones, accum.at[idxbuf.at[j]], hsems[u], add=True)
        return 0
    lax.fori_loop(0, GPW // 4, _grp, 0)

    for u in range(4):
        pltpu.make_async_copy(ones, accum.at[pl.ds(0, G)], hsems[u]).wait()

    plsc.subcore_barrier()
    pltpu.sync_copy(accum.at[pl.ds(s * PT, PT)], degp.at[c, pl.ds(s * PT, PT)])


_k1 = pl.kernel(
    _k1_body,
    out_type=jax.ShapeDtypeStruct((NC, NPAD), jnp.float32),
    mesh=plsc.VectorSubcoreMesh(core_axis_name="c", subcore_axis_name="s"),
    compiler_params=pltpu.CompilerParams(use_tc_tiling_on_sc=False),
    scratch_types=[
        pltpu.VMEM_SHARED((NPAD,), jnp.float32),
        pltpu.VMEM((GPW, G), jnp.int32),
        pltpu.VMEM((G,), jnp.float32),
        pltpu.VMEM((PT,), jnp.float32),
    ] + [pltpu.SemaphoreType.DMA] * 4,
)


def _k2_body(x_ref, w_ref, deg_ref, hs_ref):
    deg = deg_ref[0] + deg_ref[1] + 1.0
    dis = lax.rsqrt(deg)
    h = jnp.dot(x_ref[...], w_ref[...], preferred_element_type=jnp.float32)
    hs = h * dis
    hs_ref[0] = hs[:, :DH]
    hs_ref[1] = hs[:, DH:]


def _k2(x, w, degp2):
    return pl.pallas_call(
        _k2_body,
        grid=(NBLK,),
        in_specs=[
            pl.BlockSpec((ROWBLK, D_IN), lambda i: (i, 0)),
            pl.BlockSpec((D_IN, D_OUT), lambda i: (0, 0)),
            pl.BlockSpec((NC, ROWBLK, 1), lambda i: (0, i, 0)),
        ],
        out_specs=pl.BlockSpec((NC, ROWBLK, DH), lambda i: (0, i, 0)),
        out_shape=jax.ShapeDtypeStruct((NC, NPAD, DH), jnp.float32),
    )(x, w, degp2)


def _k3_body(hs2, src2, dst2, agg, accum, srcbuf, dstbuf, rows,
             g0, g1, i0, i1, i2, i3, ssem):
    c = lax.axis_index("c")
    s = lax.axis_index("s")
    off = (c * NPAD).astype(jnp.int32)
    gsems = [g0, g1]
    isems = [i0, i1, i2, i3]

    pltpu.sync_copy(hs2.at[pl.ds(c * NPAD + s * PT, PT)],
                    accum.at[pl.ds(s * PT, PT)])
    plsc.subcore_barrier()

    def _load_idx(ch, q):
        pltpu.async_copy(src2.at[pl.ds(s * GPT + ch * CH, CH)], srcbuf.at[q],
                         isems[q])
        pltpu.async_copy(dst2.at[pl.ds(s * GPT + ch * CH, CH)], dstbuf.at[q],
                         isems[q])

    def _drain_idx(q):
        pltpu.make_async_copy(src2.at[pl.ds(0, CH)], srcbuf.at[q],
                              isems[q]).wait()
        pltpu.make_async_copy(src2.at[pl.ds(0, CH)], dstbuf.at[q],
                              isems[q]).wait()

    def _bias(q):
        for r in range(CH):
            for k in range(G // L):
                sl = (q, r, pl.ds(k * L, L))
                srcbuf[sl] = srcbuf[sl] + off

    def _start_gathers(b, q):
        for r in range(CH):
            pltpu.async_copy(hs2.at[srcbuf.at[q, r]], rows.at[b, r], gsems[b])

    def _drain_rows(sem, b):
        for r in range(CH):
            pltpu.make_async_copy(hs2.at[pl.ds(0, G)], rows.at[b, r],
                                  sem).wait()

    for q in range(NSLOT):
        _load_idx(q, q)
    for b in range(2):
        _drain_idx(b)
        _bias(b)
        _start_gathers(b, b)

    def _step(i, _):
        ch4 = i * NSLOT
        for u in range(NSLOT):
            ch = ch4 + u
            b = u % 2
            _drain_rows(gsems[b], b)
            for r in range(CH):
                pltpu.async_copy(rows.at[b, r],
                                 accum.at[dstbuf.at[u, r]], ssem, add=True)

            q2 = (u + 2) % NSLOT

            @pl.when(ch + 2 < NCH)
            def _():
                _drain_idx(q2)
                _bias(q2)

            _drain_rows(ssem, b)

            @pl.when(ch + NSLOT < NCH)
            def _():
                _load_idx(ch + NSLOT, u)

            @pl.when(ch + 2 < NCH)
            def _():
                _start_gathers(b, q2)
        return 0

    lax.fori_loop(0, NCH // NSLOT, _step, 0)

    plsc.subcore_barrier()
    pltpu.sync_copy(accum.at[pl.ds(s * PT, PT)], agg.at[c, pl.ds(s * PT, PT)])


_k3 = pl.kernel(
    _k3_body,
    out_type=jax.ShapeDtypeStruct((NC, NPAD, DH), jnp.float32),
    mesh=plsc.VectorSubcoreMesh(core_axis_name="c", subcore_axis_name="s"),
    compiler_params=pltpu.CompilerParams(use_tc_tiling_on_sc=False),
    scratch_types=[
        pltpu.VMEM_SHARED((NPAD, DH), jnp.float32),
        pltpu.VMEM((NSLOT, CH, G), jnp.int32),
        pltpu.VMEM((NSLOT, CH, G), jnp.int32),
        pltpu.VMEM((2, CH, G, DH), jnp.float32),
        pltpu.SemaphoreType.DMA,
        pltpu.SemaphoreType.DMA,
        pltpu.SemaphoreType.DMA,
        pltpu.SemaphoreType.DMA,
        pltpu.SemaphoreType.DMA,
        pltpu.SemaphoreType.DMA,
        pltpu.SemaphoreType.DMA,
    ],
)


def _k4_body(agg_ref, deg_ref, b_ref, out_ref):
    dis = lax.rsqrt(deg_ref[0] + deg_ref[1] + 1.0)
    out_ref[:, :DH] = agg_ref[0] * dis + b_ref[:, :DH]
    out_ref[:, DH:] = agg_ref[1] * dis + b_ref[:, DH:]


def _k4(agg, degp2, b2):
    return pl.pallas_call(
        _k4_body,
        grid=(NBLK4,),
        in_specs=[
            pl.BlockSpec((NC, RB4, DH), lambda i: (0, i, 0)),
            pl.BlockSpec((NC, RB4, 1), lambda i: (0, i, 0)),
            pl.BlockSpec((1, D_OUT), lambda i: (0, 0)),
        ],
        out_specs=pl.BlockSpec((RB4, D_OUT), lambda i: (i, 0)),
        out_shape=jax.ShapeDtypeStruct((N, D_OUT), jnp.float32),
    )(agg, degp2, b2)


@jax.jit
def kernel(x, edge_index, W, b):
    src = edge_index[0]
    dst = edge_index[1]
    npad_e = EPAD - E
    ar = jnp.arange(npad_e, dtype=jnp.int32)
    srcp = jnp.concatenate([src, ar % N])
    dstp = jnp.concatenate([dst, N + ar % (NPAD - N)])
    src2 = srcp.reshape(NS * GPT, G)
    dst2 = dstp.reshape(NS * GPT, G)

    b2 = b.reshape(1, D_OUT)

    degp = _k1(dst2)
    degp2 = degp.reshape(NC, NPAD, 1)
    hs = _k2(x, W, degp2)
    hs2 = hs.reshape(NC * NPAD, DH)
    agg = _k3(hs2, src2, dst2)
    out = _k4(agg, degp2, b2)
    return out
